# Initial kernel scaffold; baseline (speedup 1.0000x reference)
import jax
import jax.numpy as jnp
from jax.experimental import pallas as pl


def kernel(x, edge_index, edges, edges_type, comp1, bases1, root1, bias1, comp2, bases2, root2, bias2):
    raise NotImplementedError("write your pallas kernel here")


# speedup vs baseline: 9.2745x; 9.2745x over previous
import jax
import jax.numpy as jnp
import numpy as np
from jax import lax
from jax.experimental import pallas as pl
from jax.experimental.pallas import tpu as pltpu
from jax.experimental.pallas import tpu_sc as plsc

N = 10000
E = 320000
R = 2
NB = 5
F_IN = 128
H = 16
C = 40
CP = 48
ROWS = R * N
NTILE = 16
NCORE = 2
NW = NCORE * NTILE
ACC_ROWS = 20480
RPT = ACC_ROWS // NTILE
DUMMY_DST = 10032
CHUNK = 128
NCH = 80
EP = NW * NCH * CHUNK
EROWS = EP // CHUNK
CW = 16

NPK = N * H // 128
APK = ACC_ROWS * H // 128

_M48_NP = np.zeros((128, 8 * CP), np.float32)
for _g in range(8):
    _M48_NP[16 * _g, CP * _g:CP * (_g + 1)] = 1.0


def _dense1_body(x_ref, basesc_ref, root_ref, bias_ref, src_ref, dst_ref,
                 et_ref, comp_ref, t1_ref, xr_ref, gidx_ref, sidx_ref):
    x = x_ref[...]
    p = jnp.dot(x, basesc_ref[...], preferred_element_type=jnp.float32,
                precision=lax.Precision.HIGHEST)
    for r in range(R):
        y = comp_ref[r, 0] * p[:, 0:H]
        for b in range(1, NB):
            y = y + comp_ref[r, b] * p[:, b * H:(b + 1) * H]
        t1_ref[r * N:(r + 1) * N, :] = y
    xr_ref[...] = (jnp.dot(x, root_ref[...], preferred_element_type=jnp.float32,
                precision=lax.Precision.HIGHEST)
                   + bias_ref[...])
    et = et_ref[...]
    gidx_ref[...] = et * N + src_ref[...]
    sidx_ref[...] = et * N + dst_ref[...]


_dense1 = pl.pallas_call(
    _dense1_body,
    out_shape=[
        jax.ShapeDtypeStruct((ROWS, H), jnp.float32),
        jax.ShapeDtypeStruct((N, H), jnp.float32),
        jax.ShapeDtypeStruct((EROWS, CHUNK), jnp.int32),
        jax.ShapeDtypeStruct((EROWS, CHUNK), jnp.int32),
    ],
    in_specs=[pl.BlockSpec(memory_space=pltpu.VMEM)] * 7
    + [pl.BlockSpec(memory_space=pltpu.SMEM)],
)


def _edge1_body(t1, gidx, sidx, z16, zc, ones_h, acc_out, cnt_out,
                acc_sh, cnt_sh, gidx_v, sidx_v, rows_v, ones_v, sem):
    cid = lax.axis_index("c")
    sid = lax.axis_index("s")
    wid = cid * NTILE + sid
    pltpu.sync_copy(z16.at[pl.ds(sid * RPT, RPT)],
                    acc_sh.at[pl.ds(sid * RPT, RPT)])
    pltpu.sync_copy(zc.at[pl.ds(sid * RPT, RPT)],
                    cnt_sh.at[pl.ds(sid * RPT, RPT)])
    pltpu.sync_copy(ones_h, ones_v)
    pltpu.sync_copy(gidx.at[pl.ds(wid * NCH, NCH)], gidx_v)
    pltpu.sync_copy(sidx.at[pl.ds(wid * NCH, NCH)], sidx_v)
    plsc.subcore_barrier()

    def body(c, carry):
        pltpu.async_copy(t1.at[gidx_v.at[c]], rows_v, sem).wait()
        pltpu.sync_copy(rows_v, acc_sh.at[sidx_v.at[c]], add=True)
        pltpu.sync_copy(ones_v, cnt_sh.at[sidx_v.at[c]], add=True)
        return carry

    lax.fori_loop(0, NCH, body, 0)
    plsc.subcore_barrier()
    pltpu.sync_copy(acc_sh.at[pl.ds(sid * RPT, RPT)],
                    acc_out.at[cid, pl.ds(sid * RPT, RPT)])
    pltpu.sync_copy(cnt_sh.at[pl.ds(sid * RPT, RPT)],
                    cnt_out.at[cid, pl.ds(sid * RPT, RPT)])


_edge1 = pl.kernel(
    _edge1_body,
    out_type=[
        jax.ShapeDtypeStruct((NCORE, ACC_ROWS, H), jnp.float32),
        jax.ShapeDtypeStruct((NCORE, ACC_ROWS, CW), jnp.float32),
    ],
    mesh=plsc.VectorSubcoreMesh(core_axis_name="c", subcore_axis_name="s",
                                num_cores=NCORE, num_subcores=NTILE),
    compiler_params=pltpu.CompilerParams(use_tc_tiling_on_sc=False),
    scratch_types=[
        pltpu.VMEM_SHARED((ACC_ROWS, H), jnp.float32),
        pltpu.VMEM_SHARED((ACC_ROWS, CW), jnp.float32),
        pltpu.VMEM((NCH, CHUNK), jnp.int32),
        pltpu.VMEM((NCH, CHUNK), jnp.int32),
        pltpu.VMEM((CHUNK, H), jnp.float32),
        pltpu.VMEM((CHUNK, CW), jnp.float32),
        pltpu.SemaphoreType.DMA,
    ],
)


def _edge2_body(t2, gidx, sidx, z48, acc_out,
                acc_sh, gidx_v, sidx_v, rows_v, sem):
    cid = lax.axis_index("c")
    sid = lax.axis_index("s")
    wid = cid * NTILE + sid
    pltpu.sync_copy(z48.at[pl.ds(sid * RPT, RPT)],
                    acc_sh.at[pl.ds(sid * RPT, RPT)])
    pltpu.sync_copy(gidx.at[pl.ds(wid * NCH, NCH)], gidx_v)
    pltpu.sync_copy(sidx.at[pl.ds(wid * NCH, NCH)], sidx_v)
    plsc.subcore_barrier()

    def body(c, carry):
        pltpu.async_copy(t2.at[gidx_v.at[c]], rows_v, sem).wait()
        pltpu.sync_copy(rows_v, acc_sh.at[sidx_v.at[c]], add=True)
        return carry

    lax.fori_loop(0, NCH, body, 0)
    plsc.subcore_barrier()
    pltpu.sync_copy(acc_sh.at[pl.ds(sid * RPT, RPT)],
                    acc_out.at[cid, pl.ds(sid * RPT, RPT)])


_edge2 = pl.kernel(
    _edge2_body,
    out_type=[jax.ShapeDtypeStruct((NCORE, ACC_ROWS, CP), jnp.float32)],
    mesh=plsc.VectorSubcoreMesh(core_axis_name="c", subcore_axis_name="s",
                                num_cores=NCORE, num_subcores=NTILE),
    compiler_params=pltpu.CompilerParams(use_tc_tiling_on_sc=False),
    scratch_types=[
        pltpu.VMEM_SHARED((ACC_ROWS, CP), jnp.float32),
        pltpu.VMEM((NCH, CHUNK), jnp.int32),
        pltpu.VMEM((NCH, CHUNK), jnp.int32),
        pltpu.VMEM((CHUNK, CP), jnp.float32),
        pltpu.SemaphoreType.DMA,
    ],
)


def _combine_body(xrp_ref, p1r_ref, cntr_ref, m_ref, hp_ref, inv48_ref):
    cnt = cntr_ref[0] + cntr_ref[1]
    inv = 1.0 / jnp.maximum(cnt, 1.0)
    sp = p1r_ref[0] + p1r_ref[1]
    h = (xrp_ref[...] + sp[0:NPK] * inv[0:NPK]
         + sp[NPK:2 * NPK] * inv[NPK:2 * NPK])
    hp_ref[...] = jnp.maximum(h, 0.0)
    m = m_ref[...]
    inv48_ref[0] = jnp.dot(inv[0:NPK], m, preferred_element_type=jnp.float32,
                precision=lax.Precision.HIGHEST)
    inv48_ref[1] = jnp.dot(inv[NPK:2 * NPK], m,
                           preferred_element_type=jnp.float32,
                precision=lax.Precision.HIGHEST)


_combine = pl.pallas_call(
    _combine_body,
    out_shape=[
        jax.ShapeDtypeStruct((NPK, 128), jnp.float32),
        jax.ShapeDtypeStruct((R, NPK, 8 * CP), jnp.float32),
    ],
)


def _dense2_body(h_ref, basesc2_ref, root2p_ref, bias2p_ref, comp2_ref,
                 t2_ref, hr2_ref):
    h = h_ref[...]
    p2 = jnp.dot(h, basesc2_ref[...], preferred_element_type=jnp.float32,
                precision=lax.Precision.HIGHEST)
    for r in range(R):
        y = comp2_ref[r, 0] * p2[:, 0:CP]
        for b in range(1, NB):
            y = y + comp2_ref[r, b] * p2[:, b * CP:(b + 1) * CP]
        t2_ref[r * N:(r + 1) * N, :] = y
    hr2_ref[...] = (jnp.dot(h, root2p_ref[...],
                            preferred_element_type=jnp.float32,
                precision=lax.Precision.HIGHEST)
                    + bias2p_ref[...])


_dense2 = pl.pallas_call(
    _dense2_body,
    out_shape=[
        jax.ShapeDtypeStruct((ROWS, CP), jnp.float32),
        jax.ShapeDtypeStruct((N, CP), jnp.float32),
    ],
    in_specs=[pl.BlockSpec(memory_space=pltpu.VMEM)] * 4
    + [pl.BlockSpec(memory_space=pltpu.SMEM)],
)


def _final_body(hr2_ref, p2_ref, inv48_ref, out_ref):
    s = p2_ref[0] + p2_ref[1]
    o = (hr2_ref[...] + s[0:N] * inv48_ref[0]
         + s[N:2 * N] * inv48_ref[1])
    oc = o[:, 0:C]
    m = jnp.max(oc, axis=1, keepdims=True)
    lse = m + jnp.log(jnp.sum(jnp.exp(oc - m), axis=1, keepdims=True))
    out_ref[...] = oc - lse


_final = pl.pallas_call(
    _final_body,
    out_shape=jax.ShapeDtypeStruct((N, C), jnp.float32),
)


def kernel(x, edge_index, edges, edges_type, comp1, bases1, root1, bias1,
           comp2, bases2, root2, bias2):
    f32 = jnp.float32
    basesc1 = jnp.transpose(bases1, (1, 0, 2)).reshape(F_IN, NB * H)
    bases2p = jnp.pad(bases2, ((0, 0), (0, 0), (0, CP - C)))
    basesc2 = jnp.transpose(bases2p, (1, 0, 2)).reshape(H, NB * CP)
    root2p = jnp.pad(root2, ((0, 0), (0, CP - C)))
    bias2p = jnp.pad(bias2, (0, CP - C)).reshape(1, CP)
    bias1r = bias1.reshape(1, H)
    pad = EP - E
    srcp = jnp.concatenate(
        [edges[0], jnp.zeros((pad,), jnp.int32)]).reshape(EROWS, CHUNK)
    dstp = jnp.concatenate(
        [edges[1], jnp.full((pad,), DUMMY_DST, jnp.int32)]).reshape(EROWS, CHUNK)
    etp = jnp.concatenate(
        [edges_type, jnp.ones((pad,), jnp.int32)]).reshape(EROWS, CHUNK)
    z16 = jnp.zeros((ACC_ROWS, H), f32)
    zc = jnp.zeros((ACC_ROWS, CW), f32)
    z48 = jnp.zeros((ACC_ROWS, CP), f32)
    ones_h = jnp.ones((CHUNK, CW), f32)

    t1, xr, gidx, sidx = _dense1(x, basesc1, root1, bias1r, srcp, dstp, etp,
                                 comp1)
    p1, cntp = _edge1(t1, gidx, sidx, z16, zc, ones_h)
    hp, inv48 = _combine(xr.reshape(NPK, 128),
                         p1.reshape(NCORE, APK, 128),
                         cntp.reshape(NCORE, APK, 128),
                         jnp.asarray(_M48_NP))
    t2, hr2 = _dense2(hp.reshape(N, H), basesc2, root2p, bias2p, comp2)
    (p2,) = _edge2(t2, gidx, sidx, z48)
    return _final(hr2, p2, inv48.reshape(R, N, CP))
